# baseline (speedup 1.0000x reference)
import functools

import jax
import jax.numpy as jnp
from jax import lax
from jax.experimental import pallas as pl
from jax.experimental.pallas import tpu as pltpu
from jax.experimental.pallas import tpu_sc as plsc

N = 10000
E = 320000
DIN = 128
DHID = 128
DH2 = DHID // 2
DOUT = 16

NC, NS = 2, 16
NW = NC * NS
CHUNK = 128
EC = E // CHUNK
NPAD = 10240
RPT = NPAD // NS

NCH_W, REM_W = EC // NW, EC % NW
NCH_S, REM_S = EC // NS, EC % NS

_MESH = plsc.VectorSubcoreMesh(core_axis_name="c", subcore_axis_name="s")


@functools.partial(
    pl.kernel,
    out_type=[jax.ShapeDtypeStruct((NPAD,), jnp.float32),
              jax.ShapeDtypeStruct((NPAD,), jnp.float32)],
    mesh=_MESH,
    scratch_types=[
        pltpu.VMEM((NCH_W, CHUNK), jnp.int32),
        pltpu.VMEM((CHUNK,), jnp.int32),
        pltpu.VMEM((CHUNK,), jnp.float32),
        pltpu.SemaphoreType.DMA,
        pltpu.VMEM_SHARED((NPAD,), jnp.float32),
    ],
    compiler_params=pltpu.CompilerParams(use_tc_tiling_on_sc=False),
)
def _deg_kernel(ei_hbm, zeros_hbm, out0_hbm, out1_hbm,
                didx_all, didx_e, ones_v, sem, acc):
    c = lax.axis_index("c")
    s = lax.axis_index("s")
    w = s * NC + c
    for i in range(CHUNK // 16):
        ones_v[pl.ds(i * 16, 16)] = jnp.ones((16,), jnp.float32)
    rs = s * RPT
    pltpu.sync_copy(zeros_hbm.at[pl.ds(rs, RPT)], acc.at[pl.ds(rs, RPT)])
    plsc.subcore_barrier()
    cbase = w * NCH_W + jnp.minimum(w, REM_W)
    pltpu.sync_copy(ei_hbm.at[1, pl.ds(cbase, NCH_W)], didx_all)

    def fire(j, carry):
        pltpu.async_copy(ones_v, acc.at[didx_all.at[j]], sem, add=True)
        return carry

    lax.fori_loop(0, NCH_W, fire, None)

    @pl.when(w < REM_W)
    def _():
        pltpu.sync_copy(ei_hbm.at[1, cbase + NCH_W], didx_e)
        pltpu.async_copy(ones_v, acc.at[didx_e], sem, add=True)

    def drain(j, carry):
        pltpu.make_async_copy(ones_v, acc.at[didx_e], sem).wait()
        return carry

    lax.fori_loop(0, NCH_W, drain, None)

    @pl.when(w < REM_W)
    def _():
        pltpu.make_async_copy(ones_v, acc.at[didx_e], sem).wait()

    plsc.subcore_barrier()

    @pl.when(c == 0)
    def _():
        pltpu.sync_copy(acc.at[pl.ds(rs, RPT)], out0_hbm.at[pl.ds(rs, RPT)])

    @pl.when(c == 1)
    def _():
        pltpu.sync_copy(acc.at[pl.ds(rs, RPT)], out1_hbm.at[pl.ds(rs, RPT)])


def _edge_pipeline(tab, ei_hbm, acc,
                   sidx_all, didx_all, sidx_e, didx_e, rows, sems,
                   cbase, hsz, nstage, extra):
    u = len(rows)
    for st in range(nstage):
        sb = cbase + st * hsz
        pltpu.sync_copy(ei_hbm.at[0, pl.ds(sb, hsz)], sidx_all)
        pltpu.sync_copy(ei_hbm.at[1, pl.ds(sb, hsz)], didx_all)

        for k in range(u):
            pltpu.async_copy(tab.at[sidx_all.at[k]], rows[k], sems[k])

        def body(jj, carry):
            jb = jj * u
            for k in range(u):
                j = jb + k
                pltpu.make_async_copy(tab.at[sidx_all.at[j]],
                                      rows[k], sems[k]).wait()
                pltpu.sync_copy(rows[k], acc.at[didx_all.at[j]], add=True)

                @pl.when(j + u < hsz)
                def _():
                    pltpu.async_copy(tab.at[sidx_all.at[j + u]],
                                     rows[k], sems[k])

            return carry

        lax.fori_loop(0, hsz // u, body, None)

    @pl.when(extra)
    def _():
        pltpu.sync_copy(ei_hbm.at[0, cbase + nstage * hsz], sidx_e)
        pltpu.sync_copy(ei_hbm.at[1, cbase + nstage * hsz], didx_e)
        pltpu.async_copy(tab.at[sidx_e], rows[0], sems[0]).wait()
        pltpu.sync_copy(rows[0], acc.at[didx_e], add=True)


@functools.partial(
    pl.kernel,
    out_type=[jax.ShapeDtypeStruct((NPAD, DH2), jnp.float32),
              jax.ShapeDtypeStruct((NPAD, DH2), jnp.float32)],
    mesh=_MESH,
    scratch_types=[
        pltpu.VMEM((52, CHUNK), jnp.int32),
        pltpu.VMEM((52, CHUNK), jnp.int32),
        pltpu.VMEM((CHUNK,), jnp.int32),
        pltpu.VMEM((CHUNK,), jnp.int32),
        pltpu.VMEM((CHUNK, DH2), jnp.float32),
        pltpu.VMEM((CHUNK, DH2), jnp.float32),
        pltpu.VMEM((CHUNK, DH2), jnp.float32),
        pltpu.VMEM((CHUNK, DH2), jnp.float32),
        pltpu.SemaphoreType.DMA,
        pltpu.SemaphoreType.DMA,
        pltpu.SemaphoreType.DMA,
        pltpu.SemaphoreType.DMA,
        pltpu.VMEM_SHARED((NPAD, DH2), jnp.float32),
        pltpu.VMEM_SHARED((NPAD, DH2), jnp.float32),
    ],
    compiler_params=pltpu.CompilerParams(use_tc_tiling_on_sc=False),
)
def _prop128(tabl_hbm, tabr_hbm, ei_hbm, outl_hbm, outr_hbm,
             sidx_all, didx_all, sidx_e, didx_e,
             rows0, rows1, rows2, rows3, sem0, sem1, sem2, sem3,
             tab, acc):
    c = lax.axis_index("c")
    s = lax.axis_index("s")
    rs = s * RPT

    @pl.when(c == 0)
    def _():
        pltpu.sync_copy(tabl_hbm.at[pl.ds(rs, RPT)], tab.at[pl.ds(rs, RPT)])
        pltpu.sync_copy(tabl_hbm.at[pl.ds(rs, RPT)], acc.at[pl.ds(rs, RPT)])

    @pl.when(c == 1)
    def _():
        pltpu.sync_copy(tabr_hbm.at[pl.ds(rs, RPT)], tab.at[pl.ds(rs, RPT)])
        pltpu.sync_copy(tabr_hbm.at[pl.ds(rs, RPT)], acc.at[pl.ds(rs, RPT)])

    plsc.subcore_barrier()
    cbase = s * NCH_S + jnp.minimum(s, REM_S)
    _edge_pipeline(tab, ei_hbm, acc,
                   sidx_all, didx_all, sidx_e, didx_e,
                   [rows0, rows1, rows2, rows3], [sem0, sem1, sem2, sem3],
                   cbase, 52, NCH_S // 52, s < REM_S)
    plsc.subcore_barrier()

    @pl.when(c == 0)
    def _():
        pltpu.sync_copy(acc.at[pl.ds(rs, RPT)], outl_hbm.at[pl.ds(rs, RPT)])

    @pl.when(c == 1)
    def _():
        pltpu.sync_copy(acc.at[pl.ds(rs, RPT)], outr_hbm.at[pl.ds(rs, RPT)])


@functools.partial(
    pl.kernel,
    out_type=[jax.ShapeDtypeStruct((NPAD, DOUT), jnp.float32),
              jax.ShapeDtypeStruct((NPAD, DOUT), jnp.float32)],
    mesh=_MESH,
    scratch_types=[
        pltpu.VMEM((NCH_W, CHUNK), jnp.int32),
        pltpu.VMEM((NCH_W, CHUNK), jnp.int32),
        pltpu.VMEM((CHUNK,), jnp.int32),
        pltpu.VMEM((CHUNK,), jnp.int32),
        pltpu.VMEM((CHUNK, DOUT), jnp.float32),
        pltpu.VMEM((CHUNK, DOUT), jnp.float32),
        pltpu.SemaphoreType.DMA,
        pltpu.SemaphoreType.DMA,
        pltpu.VMEM_SHARED((NPAD, DOUT), jnp.float32),
    ],
    compiler_params=pltpu.CompilerParams(use_tc_tiling_on_sc=False),
)
def _prop16(tab_hbm, ei_hbm, zeros_hbm, out0_hbm, out1_hbm,
            sidx_all, didx_all, sidx_e, didx_e, rows0, rows1, sem0, sem1,
            acc):
    c = lax.axis_index("c")
    s = lax.axis_index("s")
    w = s * NC + c
    rs = s * RPT

    @pl.when(c == 0)
    def _():
        pltpu.sync_copy(tab_hbm.at[pl.ds(rs, RPT)], acc.at[pl.ds(rs, RPT)])

    @pl.when(c == 1)
    def _():
        pltpu.sync_copy(zeros_hbm.at[pl.ds(rs, RPT)], acc.at[pl.ds(rs, RPT)])

    plsc.subcore_barrier()
    cbase = w * NCH_W + jnp.minimum(w, REM_W)
    _edge_pipeline(tab_hbm, ei_hbm, acc,
                   sidx_all, didx_all, sidx_e, didx_e, [rows0, rows1],
                   [sem0, sem1], cbase, NCH_W, 1, w < REM_W)
    plsc.subcore_barrier()

    @pl.when(c == 0)
    def _():
        pltpu.sync_copy(acc.at[pl.ds(rs, RPT)], out0_hbm.at[pl.ds(rs, RPT)])

    @pl.when(c == 1)
    def _():
        pltpu.sync_copy(acc.at[pl.ds(rs, RPT)], out1_hbm.at[pl.ds(rs, RPT)])


BLK = 2000


def _scale_mm_body(x_ref, w_ref, d0, d1, hl_ref, hr_ref, dinv_ref):
    deg = d0[...] + d1[...] + 1.0
    dinv = lax.rsqrt(deg)
    h = jnp.dot(x_ref[...], w_ref[...], preferred_element_type=jnp.float32)
    hhat = h * dinv
    hl_ref[...] = hhat[:, :DH2]
    hr_ref[...] = hhat[:, DH2:]
    dinv_ref[...] = dinv


def _scale_mm(x, w, deg0, deg1):
    return pl.pallas_call(
        _scale_mm_body,
        grid=(N // BLK,),
        in_specs=[
            pl.BlockSpec((BLK, DIN), lambda i: (i, 0)),
            pl.BlockSpec((DIN, DHID), lambda i: (0, 0)),
            pl.BlockSpec((BLK, 1), lambda i: (i, 0)),
            pl.BlockSpec((BLK, 1), lambda i: (i, 0)),
        ],
        out_specs=[
            pl.BlockSpec((BLK, DH2), lambda i: (i, 0)),
            pl.BlockSpec((BLK, DH2), lambda i: (i, 0)),
            pl.BlockSpec((BLK, 1), lambda i: (i, 0)),
        ],
        out_shape=[
            jax.ShapeDtypeStruct((NPAD, DH2), jnp.float32),
            jax.ShapeDtypeStruct((NPAD, DH2), jnp.float32),
            jax.ShapeDtypeStruct((N, 1), jnp.float32),
        ],
    )(x, w, deg0, deg1)


def _mid_body(al, ar, dinv_ref, wl_ref, wr_ref, b1_ref, hhat2_ref):
    dinv = dinv_ref[...]
    outl = al[...] * dinv + b1_ref[:, :DH2]
    outr = ar[...] * dinv + b1_ref[:, DH2:]
    h2 = (jnp.dot(outl, wl_ref[...], preferred_element_type=jnp.float32)
          + jnp.dot(outr, wr_ref[...], preferred_element_type=jnp.float32))
    hhat2_ref[...] = h2 * dinv


def _mid(accl, accr, dinv, w2, b1row):
    return pl.pallas_call(
        _mid_body,
        grid=(N // BLK,),
        in_specs=[
            pl.BlockSpec((BLK, DH2), lambda i: (i, 0)),
            pl.BlockSpec((BLK, DH2), lambda i: (i, 0)),
            pl.BlockSpec((BLK, 1), lambda i: (i, 0)),
            pl.BlockSpec((DH2, DOUT), lambda i: (0, 0)),
            pl.BlockSpec((DH2, DOUT), lambda i: (1, 0)),
            pl.BlockSpec((1, DHID), lambda i: (0, 0)),
        ],
        out_specs=pl.BlockSpec((BLK, DOUT), lambda i: (i, 0)),
        out_shape=jax.ShapeDtypeStruct((NPAD, DOUT), jnp.float32),
    )(accl, accr, dinv, w2, w2, b1row)


def _final_body(a0, a1, dinv_ref, b2_ref, out_ref):
    out_ref[...] = (a0[...] + a1[...]) * dinv_ref[...] + b2_ref[...]


def _final(acc2a, acc2b, dinv, b2row):
    return pl.pallas_call(
        _final_body,
        grid=(N // BLK,),
        in_specs=[
            pl.BlockSpec((BLK, DOUT), lambda i: (i, 0)),
            pl.BlockSpec((BLK, DOUT), lambda i: (i, 0)),
            pl.BlockSpec((BLK, 1), lambda i: (i, 0)),
            pl.BlockSpec((1, DOUT), lambda i: (0, 0)),
        ],
        out_specs=pl.BlockSpec((BLK, DOUT), lambda i: (i, 0)),
        out_shape=jax.ShapeDtypeStruct((N, DOUT), jnp.float32),
    )(acc2a, acc2b, dinv, b2row)


def kernel(x, edge_index, W1, b1, W2, b2):
    ei = edge_index.astype(jnp.int32).reshape(2, EC, CHUNK)
    zeros_n = jnp.zeros((NPAD,), jnp.float32)
    zeros_o = jnp.zeros((NPAD, DOUT), jnp.float32)

    dega, degb = _deg_kernel(ei, zeros_n)
    hhatl, hhatr, dinv = _scale_mm(x, W1, dega.reshape(NPAD, 1),
                                   degb.reshape(NPAD, 1))
    accl, accr = _prop128(hhatl, hhatr, ei)
    hhat2 = _mid(accl, accr, dinv, W2, b1.reshape(1, DHID))
    acc2a, acc2b = _prop16(hhat2, ei, zeros_o)
    return _final(acc2a, acc2b, dinv, b2.reshape(1, DOUT))


# speedup vs baseline: 1.0139x; 1.0139x over previous
import functools

import jax
import jax.numpy as jnp
from jax import lax
from jax.experimental import pallas as pl
from jax.experimental.pallas import tpu as pltpu
from jax.experimental.pallas import tpu_sc as plsc

N = 10000
E = 320000
DIN = 128
DHID = 128
DH2 = DHID // 2
DOUT = 16

NC, NS = 2, 16
NW = NC * NS
CHUNK = 128
EC = E // CHUNK
NPAD = 10240
RPT = NPAD // NS

NCH_W, REM_W = EC // NW, EC % NW
NCH_S, REM_S = EC // NS, EC % NS

_MESH = plsc.VectorSubcoreMesh(core_axis_name="c", subcore_axis_name="s")


@functools.partial(
    pl.kernel,
    out_type=[jax.ShapeDtypeStruct((NPAD,), jnp.float32),
              jax.ShapeDtypeStruct((NPAD,), jnp.float32)],
    mesh=_MESH,
    scratch_types=[
        pltpu.VMEM((NCH_W, CHUNK), jnp.int32),
        pltpu.VMEM((CHUNK,), jnp.int32),
        pltpu.VMEM((CHUNK,), jnp.float32),
        pltpu.SemaphoreType.DMA,
        pltpu.VMEM_SHARED((NPAD,), jnp.float32),
    ],
    compiler_params=pltpu.CompilerParams(use_tc_tiling_on_sc=False),
)
def _deg_kernel(ei_hbm, zeros_hbm, out0_hbm, out1_hbm,
                didx_all, didx_e, ones_v, sem, acc):
    c = lax.axis_index("c")
    s = lax.axis_index("s")
    w = s * NC + c
    for i in range(CHUNK // 16):
        ones_v[pl.ds(i * 16, 16)] = jnp.ones((16,), jnp.float32)
    rs = s * RPT
    pltpu.sync_copy(zeros_hbm.at[pl.ds(rs, RPT)], acc.at[pl.ds(rs, RPT)])
    plsc.subcore_barrier()
    cbase = w * NCH_W + jnp.minimum(w, REM_W)
    pltpu.sync_copy(ei_hbm.at[1, pl.ds(cbase, NCH_W)], didx_all)

    def fire(j, carry):
        pltpu.async_copy(ones_v, acc.at[didx_all.at[j]], sem, add=True)
        return carry

    lax.fori_loop(0, NCH_W, fire, None)

    @pl.when(w < REM_W)
    def _():
        pltpu.sync_copy(ei_hbm.at[1, cbase + NCH_W], didx_e)
        pltpu.async_copy(ones_v, acc.at[didx_e], sem, add=True)

    def drain(j, carry):
        pltpu.make_async_copy(ones_v, acc.at[didx_e], sem).wait()
        return carry

    lax.fori_loop(0, NCH_W, drain, None)

    @pl.when(w < REM_W)
    def _():
        pltpu.make_async_copy(ones_v, acc.at[didx_e], sem).wait()

    plsc.subcore_barrier()

    @pl.when(c == 0)
    def _():
        pltpu.sync_copy(acc.at[pl.ds(rs, RPT)], out0_hbm.at[pl.ds(rs, RPT)])

    @pl.when(c == 1)
    def _():
        pltpu.sync_copy(acc.at[pl.ds(rs, RPT)], out1_hbm.at[pl.ds(rs, RPT)])


HSZ = NCH_W


def _edge_pipeline(tab, ei_hbm, acc,
                   sidx_all, didx_all, sidx_e, didx_e, rows0, rows1,
                   sem0, sem1, cbase, nstage, extra):
    for st in range(nstage):
        sb = cbase + st * HSZ
        pltpu.sync_copy(ei_hbm.at[0, pl.ds(sb, HSZ)], sidx_all)
        pltpu.sync_copy(ei_hbm.at[1, pl.ds(sb, HSZ)], didx_all)

        pltpu.async_copy(tab.at[sidx_all.at[0]], rows0, sem0)

        def body(jj, carry):
            j0 = 2 * jj
            j1 = j0 + 1
            pltpu.async_copy(tab.at[sidx_all.at[j1]], rows1, sem1)
            pltpu.make_async_copy(tab.at[sidx_all.at[j0]],
                                  rows0, sem0).wait()
            pltpu.sync_copy(rows0, acc.at[didx_all.at[j0]], add=True)

            @pl.when(jj < HSZ // 2 - 1)
            def _():
                pltpu.async_copy(tab.at[sidx_all.at[j0 + 2]], rows0, sem0)

            pltpu.make_async_copy(tab.at[sidx_all.at[j1]],
                                  rows1, sem1).wait()
            pltpu.sync_copy(rows1, acc.at[didx_all.at[j1]], add=True)
            return carry

        lax.fori_loop(0, HSZ // 2, body, None)

    @pl.when(extra)
    def _():
        pltpu.sync_copy(ei_hbm.at[0, cbase + nstage * HSZ], sidx_e)
        pltpu.sync_copy(ei_hbm.at[1, cbase + nstage * HSZ], didx_e)
        pltpu.async_copy(tab.at[sidx_e], rows0, sem0).wait()
        pltpu.sync_copy(rows0, acc.at[didx_e], add=True)


@functools.partial(
    pl.kernel,
    out_type=[jax.ShapeDtypeStruct((NPAD, DH2), jnp.float32),
              jax.ShapeDtypeStruct((NPAD, DH2), jnp.float32)],
    mesh=_MESH,
    scratch_types=[
        pltpu.VMEM((HSZ, CHUNK), jnp.int32),
        pltpu.VMEM((HSZ, CHUNK), jnp.int32),
        pltpu.VMEM((CHUNK,), jnp.int32),
        pltpu.VMEM((CHUNK,), jnp.int32),
        pltpu.VMEM((CHUNK, DH2), jnp.float32),
        pltpu.VMEM((CHUNK, DH2), jnp.float32),
        pltpu.SemaphoreType.DMA,
        pltpu.SemaphoreType.DMA,
        pltpu.VMEM_SHARED((NPAD, DH2), jnp.float32),
        pltpu.VMEM_SHARED((NPAD, DH2), jnp.float32),
    ],
    compiler_params=pltpu.CompilerParams(use_tc_tiling_on_sc=False),
)
def _prop128(tabl_hbm, tabr_hbm, ei_hbm, outl_hbm, outr_hbm,
             sidx_all, didx_all, sidx_e, didx_e, rows0, rows1, sem0, sem1,
             tab, acc):
    c = lax.axis_index("c")
    s = lax.axis_index("s")
    rs = s * RPT

    @pl.when(c == 0)
    def _():
        pltpu.sync_copy(tabl_hbm.at[pl.ds(rs, RPT)], tab.at[pl.ds(rs, RPT)])
        pltpu.sync_copy(tabl_hbm.at[pl.ds(rs, RPT)], acc.at[pl.ds(rs, RPT)])

    @pl.when(c == 1)
    def _():
        pltpu.sync_copy(tabr_hbm.at[pl.ds(rs, RPT)], tab.at[pl.ds(rs, RPT)])
        pltpu.sync_copy(tabr_hbm.at[pl.ds(rs, RPT)], acc.at[pl.ds(rs, RPT)])

    plsc.subcore_barrier()
    cbase = s * NCH_S + jnp.minimum(s, REM_S)
    _edge_pipeline(tab, ei_hbm, acc,
                   sidx_all, didx_all, sidx_e, didx_e, rows0, rows1,
                   sem0, sem1, cbase, NCH_S // HSZ, s < REM_S)
    plsc.subcore_barrier()

    @pl.when(c == 0)
    def _():
        pltpu.sync_copy(acc.at[pl.ds(rs, RPT)], outl_hbm.at[pl.ds(rs, RPT)])

    @pl.when(c == 1)
    def _():
        pltpu.sync_copy(acc.at[pl.ds(rs, RPT)], outr_hbm.at[pl.ds(rs, RPT)])


@functools.partial(
    pl.kernel,
    out_type=[jax.ShapeDtypeStruct((NPAD, DOUT), jnp.float32),
              jax.ShapeDtypeStruct((NPAD, DOUT), jnp.float32)],
    mesh=_MESH,
    scratch_types=[
        pltpu.VMEM((NCH_W, CHUNK), jnp.int32),
        pltpu.VMEM((NCH_W, CHUNK), jnp.int32),
        pltpu.VMEM((CHUNK,), jnp.int32),
        pltpu.VMEM((CHUNK,), jnp.int32),
        pltpu.VMEM((CHUNK, DOUT), jnp.float32),
        pltpu.VMEM((CHUNK, DOUT), jnp.float32),
        pltpu.SemaphoreType.DMA,
        pltpu.SemaphoreType.DMA,
        pltpu.VMEM_SHARED((NPAD, DOUT), jnp.float32),
    ],
    compiler_params=pltpu.CompilerParams(use_tc_tiling_on_sc=False),
)
def _prop16(tab_hbm, ei_hbm, zeros_hbm, out0_hbm, out1_hbm,
            sidx_all, didx_all, sidx_e, didx_e, rows0, rows1, sem0, sem1,
            acc):
    c = lax.axis_index("c")
    s = lax.axis_index("s")
    w = s * NC + c
    rs = s * RPT

    @pl.when(c == 0)
    def _():
        pltpu.sync_copy(tab_hbm.at[pl.ds(rs, RPT)], acc.at[pl.ds(rs, RPT)])

    @pl.when(c == 1)
    def _():
        pltpu.sync_copy(zeros_hbm.at[pl.ds(rs, RPT)], acc.at[pl.ds(rs, RPT)])

    plsc.subcore_barrier()
    cbase = w * NCH_W + jnp.minimum(w, REM_W)
    _edge_pipeline(tab_hbm, ei_hbm, acc,
                   sidx_all, didx_all, sidx_e, didx_e, rows0, rows1,
                   sem0, sem1, cbase, 1, w < REM_W)
    plsc.subcore_barrier()

    @pl.when(c == 0)
    def _():
        pltpu.sync_copy(acc.at[pl.ds(rs, RPT)], out0_hbm.at[pl.ds(rs, RPT)])

    @pl.when(c == 1)
    def _():
        pltpu.sync_copy(acc.at[pl.ds(rs, RPT)], out1_hbm.at[pl.ds(rs, RPT)])


BLK = 2000


def _scale_mm_body(x_ref, w_ref, d0, d1, hl_ref, hr_ref, dinv_ref):
    deg = d0[...] + d1[...] + 1.0
    dinv = lax.rsqrt(deg)
    h = jnp.dot(x_ref[...], w_ref[...], preferred_element_type=jnp.float32)
    hhat = h * dinv
    hl_ref[...] = hhat[:, :DH2]
    hr_ref[...] = hhat[:, DH2:]
    dinv_ref[...] = dinv


def _scale_mm(x, w, deg0, deg1):
    return pl.pallas_call(
        _scale_mm_body,
        grid=(N // BLK,),
        in_specs=[
            pl.BlockSpec((BLK, DIN), lambda i: (i, 0)),
            pl.BlockSpec((DIN, DHID), lambda i: (0, 0)),
            pl.BlockSpec((BLK, 1), lambda i: (i, 0)),
            pl.BlockSpec((BLK, 1), lambda i: (i, 0)),
        ],
        out_specs=[
            pl.BlockSpec((BLK, DH2), lambda i: (i, 0)),
            pl.BlockSpec((BLK, DH2), lambda i: (i, 0)),
            pl.BlockSpec((BLK, 1), lambda i: (i, 0)),
        ],
        out_shape=[
            jax.ShapeDtypeStruct((NPAD, DH2), jnp.float32),
            jax.ShapeDtypeStruct((NPAD, DH2), jnp.float32),
            jax.ShapeDtypeStruct((N, 1), jnp.float32),
        ],
    )(x, w, deg0, deg1)


def _mid_body(al, ar, dinv_ref, wl_ref, wr_ref, b1_ref, hhat2_ref):
    dinv = dinv_ref[...]
    outl = al[...] * dinv + b1_ref[:, :DH2]
    outr = ar[...] * dinv + b1_ref[:, DH2:]
    h2 = (jnp.dot(outl, wl_ref[...], preferred_element_type=jnp.float32)
          + jnp.dot(outr, wr_ref[...], preferred_element_type=jnp.float32))
    hhat2_ref[...] = h2 * dinv


def _mid(accl, accr, dinv, w2, b1row):
    return pl.pallas_call(
        _mid_body,
        grid=(N // BLK,),
        in_specs=[
            pl.BlockSpec((BLK, DH2), lambda i: (i, 0)),
            pl.BlockSpec((BLK, DH2), lambda i: (i, 0)),
            pl.BlockSpec((BLK, 1), lambda i: (i, 0)),
            pl.BlockSpec((DH2, DOUT), lambda i: (0, 0)),
            pl.BlockSpec((DH2, DOUT), lambda i: (1, 0)),
            pl.BlockSpec((1, DHID), lambda i: (0, 0)),
        ],
        out_specs=pl.BlockSpec((BLK, DOUT), lambda i: (i, 0)),
        out_shape=jax.ShapeDtypeStruct((NPAD, DOUT), jnp.float32),
    )(accl, accr, dinv, w2, w2, b1row)


def _final_body(a0, a1, dinv_ref, b2_ref, out_ref):
    out_ref[...] = (a0[...] + a1[...]) * dinv_ref[...] + b2_ref[...]


def _final(acc2a, acc2b, dinv, b2row):
    return pl.pallas_call(
        _final_body,
        grid=(N // BLK,),
        in_specs=[
            pl.BlockSpec((BLK, DOUT), lambda i: (i, 0)),
            pl.BlockSpec((BLK, DOUT), lambda i: (i, 0)),
            pl.BlockSpec((BLK, 1), lambda i: (i, 0)),
            pl.BlockSpec((1, DOUT), lambda i: (0, 0)),
        ],
        out_specs=pl.BlockSpec((BLK, DOUT), lambda i: (i, 0)),
        out_shape=jax.ShapeDtypeStruct((N, DOUT), jnp.float32),
    )(acc2a, acc2b, dinv, b2row)


def kernel(x, edge_index, W1, b1, W2, b2):
    ei = edge_index.astype(jnp.int32).reshape(2, EC, CHUNK)
    zeros_n = jnp.zeros((NPAD,), jnp.float32)
    zeros_o = jnp.zeros((NPAD, DOUT), jnp.float32)

    dega, degb = _deg_kernel(ei, zeros_n)
    hhatl, hhatr, dinv = _scale_mm(x, W1, dega.reshape(NPAD, 1),
                                   degb.reshape(NPAD, 1))
    accl, accr = _prop128(hhatl, hhatr, ei)
    hhat2 = _mid(accl, accr, dinv, W2, b1.reshape(1, DHID))
    acc2a, acc2b = _prop16(hhat2, ei, zeros_o)
    return _final(acc2a, acc2b, dinv, b2.reshape(1, DOUT))
